# Initial kernel scaffold; baseline (speedup 1.0000x reference)
import functools

import jax
import jax.numpy as jnp
from jax import lax
from jax.experimental import pallas as pl
from jax.experimental.pallas import tpu as pltpu
from jax.experimental.pallas import tpu_sc as plsc

B, C, L = 4, 2048, 4096
GROUP_SIZES = (256, 256, 512, 1024)
GROUP_STARTS = (0, 256, 512, 1024)

NC, NS = 2, 16
NW = NC * NS
CH_PER_W = C // NW
CHUNK = 8
N_CHUNKS = CH_PER_W // CHUNK

_mesh = plsc.VectorSubcoreMesh(core_axis_name="c", subcore_axis_name="s")


@functools.partial(
    pl.kernel,
    out_type=[jax.ShapeDtypeStruct((B, g, L), jnp.float32)
              for _ in range(3) for g in GROUP_SIZES],
    mesh=_mesh,
    scratch_types=[
        pltpu.VMEM((CH_PER_W,), jnp.int32),
        pltpu.VMEM((B * CH_PER_W,), jnp.int32),
        pltpu.VMEM((CHUNK, L), jnp.float32),
        pltpu.VMEM((CHUNK, L), jnp.float32),
        pltpu.SemaphoreType.DMA,
        pltpu.SemaphoreType.DMA,
    ],
)
def _sc_regroup(q_hbm, k_hbm, v_hbm, idx_hbm,
                q0, q1, q2, q3, k0, k1, k2, k3, v0, v1, v2, v3,
                idx_v, idx_all, buf0, buf1, gsem, wsem):
    wid = lax.axis_index("s") * NC + lax.axis_index("c")
    base = wid * CH_PER_W

    pltpu.sync_copy(idx_hbm.at[pl.ds(base, CH_PER_W)], idx_v)
    for b in range(B):
        for j in range(CH_PER_W // 16):
            idx_all[pl.ds(b * CH_PER_W + j * 16, 16)] = (
                idx_v[pl.ds(j * 16, 16)] + b * C)

    bufs = (buf0, buf1)
    outs = ((q0, q1, q2, q3), (k0, k1, k2, k3), (v0, v1, v2, v3))
    tabs = (q_hbm, k_hbm, v_hbm)

    for g in range(4):
        g_lo = GROUP_STARTS[g] // CH_PER_W
        g_hi = (GROUP_STARTS[g] + GROUP_SIZES[g]) // CH_PER_W

        @pl.when(jnp.logical_and(wid >= g_lo, wid < g_hi))
        def _():
            off0 = base - GROUP_STARTS[g]
            for t in range(3):
                tab = tabs[t]
                out = outs[t][g]

                def start_gather(i, slot):
                    b = i // N_CHUNKS
                    c = lax.rem(i, N_CHUNKS)
                    src = tab.at[idx_all.at[pl.ds(b * CH_PER_W + c * CHUNK,
                                                  CHUNK)]]
                    pltpu.async_copy(src, bufs[slot], gsem)

                def start_write(i, slot):
                    b = i // N_CHUNKS
                    c = lax.rem(i, N_CHUNKS)
                    pltpu.async_copy(
                        bufs[slot],
                        out.at[b, pl.ds(off0 + c * CHUNK, CHUNK)],
                        wsem)

                n_iter = B * N_CHUNKS
                start_gather(0, 0)

                def body(i, _):
                    slot = lax.rem(i, 2)

                    @pl.when(i + 1 < n_iter)
                    def _():
                        start_gather(i + 1, 1 - slot)

                    pltpu.dma_wait(gsem, bufs[slot])
                    start_write(i, slot)
                    pltpu.dma_wait(wsem, bufs[slot])
                    return 0

                lax.fori_loop(0, n_iter, body, 0)

    return ()


def kernel(query, key, value):
    channel_features = query.mean(axis=2)
    channel_mean = channel_features.mean(axis=0)
    sorted_indices = jnp.argsort(-channel_mean).astype(jnp.int32)

    qf = query.reshape(B * C, L)
    kf = key.reshape(B * C, L)
    vf = value.reshape(B * C, L)
    outs = _sc_regroup(qf, kf, vf, sorted_indices)
    return tuple(tuple(outs[t * 4:(t + 1) * 4]) for t in range(3))


# speedup vs baseline: 2.6511x; 2.6511x over previous
import functools

import jax
import jax.numpy as jnp
from jax import lax
from jax.experimental import pallas as pl
from jax.experimental.pallas import tpu as pltpu
from jax.experimental.pallas import tpu_sc as plsc

B, C, L = 4, 2048, 4096
GROUP_SIZES = (256, 256, 512, 1024)
GROUP_STARTS = (0, 256, 512, 1024)

NC, NS = 2, 16
NW = NC * NS
CH_PER_W = C // NW
CHUNK = 8
N_CHUNKS = CH_PER_W // CHUNK
N_ITER = B * N_CHUNKS

_mesh = plsc.VectorSubcoreMesh(core_axis_name="c", subcore_axis_name="s")


@functools.partial(
    pl.kernel,
    out_type=[jax.ShapeDtypeStruct((B, g, L), jnp.float32)
              for _ in range(3) for g in GROUP_SIZES],
    mesh=_mesh,
    scratch_types=[
        pltpu.VMEM((CH_PER_W,), jnp.int32),
        pltpu.VMEM((B * CH_PER_W,), jnp.int32),
        pltpu.VMEM((CHUNK, L), jnp.float32),
        pltpu.VMEM((CHUNK, L), jnp.float32),
        pltpu.SemaphoreType.DMA,
        pltpu.SemaphoreType.DMA,
        pltpu.SemaphoreType.DMA,
        pltpu.SemaphoreType.DMA,
    ],
)
def _sc_regroup(q_hbm, k_hbm, v_hbm, idx_hbm,
                q0, q1, q2, q3, k0, k1, k2, k3, v0, v1, v2, v3,
                idx_v, idx_all, buf0, buf1, gsem0, gsem1, wsem0, wsem1):
    wid = lax.axis_index("s") * NC + lax.axis_index("c")
    base = wid * CH_PER_W

    pltpu.sync_copy(idx_hbm.at[pl.ds(base, CH_PER_W)], idx_v)
    for b in range(B):
        for j in range(CH_PER_W // 16):
            idx_all[pl.ds(b * CH_PER_W + j * 16, 16)] = (
                idx_v[pl.ds(j * 16, 16)] + b * C)

    bufs = (buf0, buf1)
    gsems = (gsem0, gsem1)
    wsems = (wsem0, wsem1)
    outs = ((q0, q1, q2, q3), (k0, k1, k2, k3), (v0, v1, v2, v3))
    tabs = (q_hbm, k_hbm, v_hbm)

    for g in range(4):
        g_lo = GROUP_STARTS[g] // CH_PER_W
        g_hi = (GROUP_STARTS[g] + GROUP_SIZES[g]) // CH_PER_W

        @pl.when(jnp.logical_and(wid >= g_lo, wid < g_hi))
        def _():
            off0 = base - GROUP_STARTS[g]
            for t in range(3):
                tab = tabs[t]
                out = outs[t][g]

                def start_gather(i, slot):
                    b = i // N_CHUNKS
                    c = lax.rem(i, N_CHUNKS)
                    idx = idx_all.at[pl.ds(b * CH_PER_W + c * CHUNK, CHUNK)]
                    pltpu.async_copy(tab.at[idx], bufs[slot], gsems[slot])

                def wait_gather(slot):
                    pltpu.make_async_copy(
                        tab.at[pl.ds(0, CHUNK)], bufs[slot],
                        gsems[slot]).wait()

                def start_write(i, slot):
                    b = i // N_CHUNKS
                    c = lax.rem(i, N_CHUNKS)
                    pltpu.async_copy(
                        bufs[slot],
                        out.at[b, pl.ds(off0 + c * CHUNK, CHUNK)],
                        wsems[slot])

                def wait_write(slot):
                    pltpu.make_async_copy(
                        bufs[slot], out.at[0, pl.ds(0, CHUNK)],
                        wsems[slot]).wait()

                start_gather(0, 0)
                start_gather(1, 1)

                def body(k, _):
                    i = 2 * k
                    wait_gather(0)
                    start_write(i, 0)
                    wait_gather(1)
                    start_write(i + 1, 1)
                    wait_write(0)

                    @pl.when(i + 2 < N_ITER)
                    def _():
                        start_gather(i + 2, 0)

                    wait_write(1)

                    @pl.when(i + 3 < N_ITER)
                    def _():
                        start_gather(i + 3, 1)

                    return 0

                lax.fori_loop(0, N_ITER // 2, body, 0)

    return None


def kernel(query, key, value):
    channel_features = query.mean(axis=2)
    channel_mean = channel_features.mean(axis=0)
    sorted_indices = jnp.argsort(-channel_mean).astype(jnp.int32)

    qf = query.reshape(B * C, L)
    kf = key.reshape(B * C, L)
    vf = value.reshape(B * C, L)
    outs = _sc_regroup(qf, kf, vf, sorted_indices)
    return tuple(tuple(outs[t * 4:(t + 1) * 4]) for t in range(3))


# speedup vs baseline: 2.6844x; 1.0125x over previous
import functools

import jax
import jax.numpy as jnp
from jax import lax
from jax.experimental import pallas as pl
from jax.experimental.pallas import tpu as pltpu
from jax.experimental.pallas import tpu_sc as plsc

B, C, L = 4, 2048, 4096
GROUP_SIZES = (256, 256, 512, 1024)
GROUP_STARTS = (0, 256, 512, 1024)

NC, NS = 2, 16
NW = NC * NS
CH_PER_W = C // NW
CHUNK = 8
N_CHUNKS = CH_PER_W // CHUNK
N_ITER = B * N_CHUNKS

_mesh = plsc.VectorSubcoreMesh(core_axis_name="c", subcore_axis_name="s")


@functools.partial(
    pl.kernel,
    out_type=[jax.ShapeDtypeStruct((B, g, L), jnp.float32)
              for _ in range(3) for g in GROUP_SIZES],
    mesh=_mesh,
    scratch_types=[
        pltpu.VMEM((CH_PER_W,), jnp.int32),
        pltpu.VMEM((B * CH_PER_W,), jnp.int32),
        pltpu.VMEM((CHUNK, L), jnp.float32),
        pltpu.VMEM((CHUNK, L), jnp.float32),
        pltpu.VMEM((CHUNK, L), jnp.float32),
        pltpu.SemaphoreType.DMA,
        pltpu.SemaphoreType.DMA,
        pltpu.SemaphoreType.DMA,
        pltpu.SemaphoreType.DMA,
        pltpu.SemaphoreType.DMA,
        pltpu.SemaphoreType.DMA,
    ],
)
def _sc_regroup(q_hbm, k_hbm, v_hbm, idx_hbm,
                q0, q1, q2, q3, k0, k1, k2, k3, v0, v1, v2, v3,
                idx_v, idx_all, buf0, buf1, buf2,
                gsem0, gsem1, gsem2, wsem0, wsem1, wsem2):
    wid = lax.axis_index("s") * NC + lax.axis_index("c")
    base = wid * CH_PER_W

    pltpu.sync_copy(idx_hbm.at[pl.ds(base, CH_PER_W)], idx_v)
    for b in range(B):
        for j in range(CH_PER_W // 16):
            idx_all[pl.ds(b * CH_PER_W + j * 16, 16)] = (
                idx_v[pl.ds(j * 16, 16)] + b * C)

    bufs = (buf0, buf1, buf2)
    gsems = (gsem0, gsem1, gsem2)
    wsems = (wsem0, wsem1, wsem2)
    outs = ((q0, q1, q2, q3), (k0, k1, k2, k3), (v0, v1, v2, v3))
    tabs = (q_hbm, k_hbm, v_hbm)

    TOT = 3 * N_ITER

    def decode(i):
        if isinstance(i, int):
            return i // N_ITER, (i % N_ITER) // N_CHUNKS, i % N_CHUNKS
        t = i // N_ITER
        r = lax.rem(i, N_ITER)
        return t, r // N_CHUNKS, lax.rem(r, N_CHUNKS)

    for g in range(4):
        g_lo = GROUP_STARTS[g] // CH_PER_W
        g_hi = (GROUP_STARTS[g] + GROUP_SIZES[g]) // CH_PER_W

        @pl.when(jnp.logical_and(wid >= g_lo, wid < g_hi))
        def _():
            off0 = base - GROUP_STARTS[g]

            def start_gather(i, slot):
                t, b, c = decode(i)
                idx = idx_all.at[pl.ds(b * CH_PER_W + c * CHUNK, CHUNK)]
                if isinstance(i, int):
                    pltpu.async_copy(tabs[t].at[idx], bufs[slot],
                                     gsems[slot])
                    return
                for tt in range(3):
                    @pl.when(t == tt)
                    def _():
                        pltpu.async_copy(tabs[tt].at[idx], bufs[slot],
                                         gsems[slot])

            def wait_gather(slot):
                pltpu.make_async_copy(
                    tabs[0].at[pl.ds(0, CHUNK)], bufs[slot],
                    gsems[slot]).wait()

            def start_write(i, slot):
                t, b, c = decode(i)
                dst = (b, pl.ds(off0 + c * CHUNK, CHUNK))
                for tt in range(3):
                    @pl.when(t == tt)
                    def _():
                        pltpu.async_copy(bufs[slot],
                                         outs[tt][g].at[dst],
                                         wsems[slot])

            def wait_write(slot):
                pltpu.make_async_copy(
                    bufs[slot], outs[0][g].at[0, pl.ds(0, CHUNK)],
                    wsems[slot]).wait()

            for s in range(3):
                start_gather(s, s)

            def body(k, _):
                i = 3 * k
                for s in range(3):
                    wait_gather(s)
                    start_write(i + s, s)
                for s in range(3):
                    wait_write(s)

                    @pl.when(i + 3 + s < TOT)
                    def _():
                        start_gather(i + 3 + s, s)
                return 0

            lax.fori_loop(0, TOT // 3, body, 0)

    return None


def kernel(query, key, value):
    channel_features = query.mean(axis=2)
    channel_mean = channel_features.mean(axis=0)
    sorted_indices = jnp.argsort(-channel_mean).astype(jnp.int32)

    qf = query.reshape(B * C, L)
    kf = key.reshape(B * C, L)
    vf = value.reshape(B * C, L)
    outs = _sc_regroup(qf, kf, vf, sorted_indices)
    return tuple(tuple(outs[t * 4:(t + 1) * 4]) for t in range(3))
